# Initial kernel scaffold; baseline (speedup 1.0000x reference)
import jax
import jax.numpy as jnp
from jax.experimental import pallas as pl


def kernel(p3, p4, p5, p6, p7, targets, image_size):
    raise NotImplementedError("write your pallas kernel here")


# speedup vs baseline: 4.6814x; 4.6814x over previous
import functools

import jax
import jax.numpy as jnp
from jax.experimental import pallas as pl

B = 8
C = 85
NCLS = 80
NT = 64
SIZES = (8.0, 16.0, 32.0, 64.0, 128.0)
IMG = 1024.0
CCHUNK = 8


def _lvl_kernel(t_ref, p_ref, o_ref, *, g, bh, s, stride, lo, hi):
    f32 = jnp.float32
    step = pl.program_id(0)
    row = jax.lax.broadcasted_iota(jnp.int32, (bh, g), 0).astype(f32)
    col = jax.lax.broadcasted_iota(jnp.int32, (bh, g), 1).astype(f32)
    y = (step.astype(f32) * bh + row + 0.5) * stride
    x = (col + 0.5) * stride

    radius = s * 2.0
    best = jnp.full((bh, g), -1.0, f32)
    keep = jnp.zeros((bh, g), jnp.bool_)
    sb = jnp.zeros((bh, g), f32)
    sc = jnp.zeros((bh, g), f32)
    sx0 = jnp.zeros((bh, g), f32)
    sy0 = jnp.zeros((bh, g), f32)
    sx1 = jnp.zeros((bh, g), f32)
    sy1 = jnp.zeros((bh, g), f32)

    for t in range(NT):
        nb = t_ref[0, t, 0]
        cl = t_ref[0, t, 1]
        x0 = t_ref[0, t, 2]
        y0 = t_ref[0, t, 3]
        x1 = t_ref[0, t, 4]
        y1 = t_ref[0, t, 5]
        l = x - x0
        tt = y - y0
        r = x1 - x
        bb = y1 - y
        omin = jnp.minimum(jnp.minimum(l, tt), jnp.minimum(r, bb))
        omax = jnp.maximum(jnp.maximum(l, tt), jnp.maximum(r, bb))
        cxb = (x0 + x1) / 2.0
        cyb = (y0 + y1) / 2.0
        cmax = jnp.maximum(jnp.abs(x - cxb), jnp.abs(y - cyb))
        jc = (omax > lo) & (omax < hi) & (omin > 0.0) & (cmax < radius)
        area = (l + r) * (tt + bb)
        score = jnp.where(jc, 1e8 - area, 0.0)
        upd = score > best
        best = jnp.where(upd, score, best)
        sb = jnp.where(upd, nb, sb)
        sc = jnp.where(upd, cl, sc)
        sx0 = jnp.where(upd, x0, sx0)
        sy0 = jnp.where(upd, y0, sy0)
        sx1 = jnp.where(upd, x1, sx1)
        sy1 = jnp.where(upd, y1, sy1)
        keep = keep | jc

    bsel = sb.astype(jnp.int32)
    csel = sc.astype(jnp.int32)
    lsel = x - sx0
    tsel = y - sy0
    rsel = sx1 - x
    bsel_f = sy1 - y
    gx = lsel + sx0
    gy = tsel + sy0
    lr_min = jnp.minimum(lsel, rsel)
    lr_max = jnp.maximum(lsel, rsel)
    tb_min = jnp.minimum(tsel, bsel_f)
    tb_max = jnp.maximum(tsel, bsel_f)
    tcnt = jnp.sqrt(lr_min * tb_min / (lr_max * tb_max + 1e-10))
    tx0 = gx - lsel
    ty0 = gy - tsel
    tx1 = gx + rsel
    ty1 = gy + bsel_f

    pid = jnp.where(keep, bsel * C + (csel + 5), -1)

    facc = jnp.zeros((bh, g), f32)
    xpos = jnp.zeros((bh, g), f32)
    ps = [jnp.zeros((bh, g), f32) for _ in range(5)]
    ciota = jax.lax.broadcasted_iota(jnp.int32, (CCHUNK, 1, 1), 0)
    for b in range(B):
        bm = bsel == b
        for c in range(5):
            ps[c] = ps[c] + jnp.where(bm, p_ref[b, c], 0.0)
        pid_b = (pid - b * C)[None, :, :]
        for c0 in range(5, C, CCHUNK):
            xc = p_ref[b, c0 : c0 + CCHUNK]
            e = jnp.exp(-jnp.abs(xc))
            lg = jnp.log1p(e)
            ce0 = jnp.maximum(xc, 0.0) + lg
            p = jax.nn.sigmoid(xc)
            q = 1.0 - (1.0 - p)
            facc = facc + jnp.sum(0.75 * ce0 * q * q, axis=0)
            cm = pid_b == (ciota + c0)
            xpos = xpos + jnp.sum(jnp.where(cm, xc, 0.0), axis=0)

    e = jnp.exp(-jnp.abs(xpos))
    lg = jnp.log1p(e)
    relu = jnp.maximum(xpos, 0.0)
    p = jax.nn.sigmoid(xpos)
    ce1 = relu - xpos + lg
    om = 1.0 - p
    f1 = 0.25 * ce1 * om * om
    ce0 = relu + lg
    q = 1.0 - om
    f0 = 0.75 * ce0 * q * q
    lcls_cells = facc + jnp.where(keep, f1 - f0, 0.0)

    xo = ps[4]
    ce = jnp.maximum(xo, 0.0) - xo * tcnt + jnp.log1p(jnp.exp(-jnp.abs(xo)))
    lcnt_cells = jnp.where(keep, ce, 0.0)

    px0 = gx - ps[0] * s
    py0 = gy - ps[1] * s
    px1 = gx + ps[2] * s
    py1 = gy + ps[3] * s
    ix0 = jnp.maximum(px0, tx0)
    iy0 = jnp.maximum(py0, ty0)
    ix1 = jnp.minimum(px1, tx1)
    iy1 = jnp.minimum(py1, ty1)
    inter = jnp.clip(ix1 - ix0, 0.0) * jnp.clip(iy1 - iy0, 0.0)
    a1 = (px1 - px0) * (py1 - py0)
    a2 = (tx1 - tx0) * (ty1 - ty0)
    union = a1 + a2 - inter + 1e-9
    iou = inter / union
    cx0 = jnp.minimum(px0, tx0)
    cy0 = jnp.minimum(py0, ty0)
    cx1 = jnp.maximum(px1, tx1)
    cy1 = jnp.maximum(py1, ty1)
    cc = (cx1 - cx0) * (cy1 - cy0) + 1e-9
    giou = iou - (cc - union) / cc
    lbox_cells = jnp.where(keep, 1.0 - giou, 0.0)

    zeros = jnp.zeros((g,), f32)
    out = jnp.stack(
        [
            jnp.sum(lbox_cells, axis=0),
            jnp.sum(lcnt_cells, axis=0),
            jnp.sum(lcls_cells, axis=0),
            jnp.sum(keep.astype(f32), axis=0),
            zeros,
            zeros,
            zeros,
            zeros,
        ],
        axis=0,
    )
    o_ref[...] = out[None]


def _run_level(p, targets, li, interpret=False):
    g = p.shape[2]
    bh = min(g, 16)
    nsteps = g // bh
    s = SIZES[li]
    stride = IMG / g
    lo = 0.0 if li == 0 else s * 4.0
    hi = float("inf") if li == 4 else s * 8.0
    out = pl.pallas_call(
        functools.partial(
            _lvl_kernel, g=g, bh=bh, s=s, stride=stride, lo=lo, hi=hi
        ),
        grid=(nsteps,),
        in_specs=[
            pl.BlockSpec((1, NT, 6), lambda i: (0, 0, 0)),
            pl.BlockSpec((B, C, bh, g), lambda i: (0, 0, i, 0)),
        ],
        out_specs=pl.BlockSpec((1, 8, g), lambda i: (i, 0, 0)),
        out_shape=jax.ShapeDtypeStruct((nsteps, 8, g), jnp.float32),
        interpret=interpret,
    )(targets, p)
    return jnp.sum(out, axis=(0, 2))


def _fcos_loss_pallas(p3, p4, p5, p6, p7, targets, interpret=False):
    tg = jnp.asarray(targets, jnp.float32)
    acc = None
    for li, p in enumerate((p3, p4, p5, p6, p7)):
        part = _run_level(p, tg, li, interpret=interpret)
        acc = part if acc is None else acc + part
    n = acc[3]
    lbox = acc[0] / n
    lcnt = acc[1] / n
    lcls = acc[2] / n
    loss = lbox + lcnt + lcls
    return (loss, lbox, lcnt, lcls)


def kernel(p3, p4, p5, p6, p7, targets, image_size):
    return _fcos_loss_pallas(p3, p4, p5, p6, p7, targets)


# speedup vs baseline: 6.1336x; 1.3102x over previous
import functools

import jax
import jax.numpy as jnp
from jax.experimental import pallas as pl

B = 8
C = 85
NCLS = 80
NT = 64
SIZES = (8.0, 16.0, 32.0, 64.0, 128.0)
IMG = 1024.0
CCHUNK = 8


def _lvl_kernel(t_ref, p_ref, o_ref, *, g, bh, s, stride, lo, hi):
    f32 = jnp.float32
    step = pl.program_id(0)
    row = jax.lax.broadcasted_iota(jnp.int32, (bh, g), 0).astype(f32)
    col = jax.lax.broadcasted_iota(jnp.int32, (bh, g), 1).astype(f32)
    y = (step.astype(f32) * bh + row + 0.5) * stride
    x = (col + 0.5) * stride

    radius = s * 2.0
    best = jnp.full((bh, g), -1.0, f32)
    keep = jnp.zeros((bh, g), jnp.bool_)
    sb = jnp.zeros((bh, g), f32)
    sc = jnp.zeros((bh, g), f32)
    sx0 = jnp.zeros((bh, g), f32)
    sy0 = jnp.zeros((bh, g), f32)
    sx1 = jnp.zeros((bh, g), f32)
    sy1 = jnp.zeros((bh, g), f32)

    for t in range(NT):
        nb = t_ref[0, t, 0]
        cl = t_ref[0, t, 1]
        x0 = t_ref[0, t, 2]
        y0 = t_ref[0, t, 3]
        x1 = t_ref[0, t, 4]
        y1 = t_ref[0, t, 5]
        l = x - x0
        tt = y - y0
        r = x1 - x
        bb = y1 - y
        omin = jnp.minimum(jnp.minimum(l, tt), jnp.minimum(r, bb))
        omax = jnp.maximum(jnp.maximum(l, tt), jnp.maximum(r, bb))
        cxb = (x0 + x1) / 2.0
        cyb = (y0 + y1) / 2.0
        cmax = jnp.maximum(jnp.abs(x - cxb), jnp.abs(y - cyb))
        jc = (omin > 0.0) & (cmax < radius)
        if lo > 0.0:
            jc = jc & (omax > lo)
        if hi != float("inf"):
            jc = jc & (omax < hi)
        area = (l + r) * (tt + bb)
        score = jnp.where(jc, 1e8 - area, 0.0)
        upd = score > best
        best = jnp.where(upd, score, best)
        sb = jnp.where(upd, nb, sb)
        sc = jnp.where(upd, cl, sc)
        sx0 = jnp.where(upd, x0, sx0)
        sy0 = jnp.where(upd, y0, sy0)
        sx1 = jnp.where(upd, x1, sx1)
        sy1 = jnp.where(upd, y1, sy1)
        keep = keep | jc

    bsel = sb.astype(jnp.int32)
    csel = sc.astype(jnp.int32)
    lsel = x - sx0
    tsel = y - sy0
    rsel = sx1 - x
    bsel_f = sy1 - y
    gx = lsel + sx0
    gy = tsel + sy0
    lr_min = jnp.minimum(lsel, rsel)
    lr_max = jnp.maximum(lsel, rsel)
    tb_min = jnp.minimum(tsel, bsel_f)
    tb_max = jnp.maximum(tsel, bsel_f)
    tcnt = jnp.sqrt(lr_min * tb_min / (lr_max * tb_max + 1e-10))
    tx0 = gx - lsel
    ty0 = gy - tsel
    tx1 = gx + rsel
    ty1 = gy + bsel_f

    pid = jnp.where(keep, bsel * C + (csel + 5), -1)

    facc = jnp.zeros((bh, g), f32)
    xpos = jnp.zeros((bh, g), f32)
    ps = [jnp.zeros((bh, g), f32) for _ in range(5)]
    ciota = jax.lax.broadcasted_iota(jnp.int32, (CCHUNK, 1, 1), 0)
    for b in range(B):
        bm = bsel == b
        for c in range(5):
            ps[c] = ps[c] + jnp.where(bm, p_ref[b, c], 0.0)
        pid_b = (pid - b * C)[None, :, :]
        for c0 in range(5, C, CCHUNK):
            xc = p_ref[b, c0 : c0 + CCHUNK]
            e2 = jnp.exp(-xc)
            t = 1.0 + e2
            r = 1.0 / t
            sp = xc + jnp.log(t)
            facc = facc + jnp.sum(sp * r * r, axis=0)
            cm = pid_b == (ciota + c0)
            xpos = xpos + jnp.sum(jnp.where(cm, xc, 0.0), axis=0)

    e = jnp.exp(-jnp.abs(xpos))
    lg = jnp.log1p(e)
    relu = jnp.maximum(xpos, 0.0)
    p = jax.nn.sigmoid(xpos)
    ce1 = relu - xpos + lg
    om = 1.0 - p
    f1 = 0.25 * ce1 * om * om
    ce0 = relu + lg
    q = 1.0 - om
    f0 = 0.75 * ce0 * q * q
    lcls_cells = 0.75 * facc + jnp.where(keep, f1 - f0, 0.0)

    xo = ps[4]
    ce = jnp.maximum(xo, 0.0) - xo * tcnt + jnp.log1p(jnp.exp(-jnp.abs(xo)))
    lcnt_cells = jnp.where(keep, ce, 0.0)

    px0 = gx - ps[0] * s
    py0 = gy - ps[1] * s
    px1 = gx + ps[2] * s
    py1 = gy + ps[3] * s
    ix0 = jnp.maximum(px0, tx0)
    iy0 = jnp.maximum(py0, ty0)
    ix1 = jnp.minimum(px1, tx1)
    iy1 = jnp.minimum(py1, ty1)
    inter = jnp.clip(ix1 - ix0, 0.0) * jnp.clip(iy1 - iy0, 0.0)
    a1 = (px1 - px0) * (py1 - py0)
    a2 = (tx1 - tx0) * (ty1 - ty0)
    union = a1 + a2 - inter + 1e-9
    iou = inter / union
    cx0 = jnp.minimum(px0, tx0)
    cy0 = jnp.minimum(py0, ty0)
    cx1 = jnp.maximum(px1, tx1)
    cy1 = jnp.maximum(py1, ty1)
    cc = (cx1 - cx0) * (cy1 - cy0) + 1e-9
    giou = iou - (cc - union) / cc
    lbox_cells = jnp.where(keep, 1.0 - giou, 0.0)

    zeros = jnp.zeros((g,), f32)
    out = jnp.stack(
        [
            jnp.sum(lbox_cells, axis=0),
            jnp.sum(lcnt_cells, axis=0),
            jnp.sum(lcls_cells, axis=0),
            jnp.sum(keep.astype(f32), axis=0),
            zeros,
            zeros,
            zeros,
            zeros,
        ],
        axis=0,
    )
    o_ref[...] = out[None]


def _run_level(p, targets, li, interpret=False):
    g = p.shape[2]
    bh = min(g, 16)
    nsteps = g // bh
    s = SIZES[li]
    stride = IMG / g
    lo = 0.0 if li == 0 else s * 4.0
    hi = float("inf") if li == 4 else s * 8.0
    out = pl.pallas_call(
        functools.partial(
            _lvl_kernel, g=g, bh=bh, s=s, stride=stride, lo=lo, hi=hi
        ),
        grid=(nsteps,),
        in_specs=[
            pl.BlockSpec((1, NT, 6), lambda i: (0, 0, 0)),
            pl.BlockSpec((B, C, bh, g), lambda i: (0, 0, i, 0)),
        ],
        out_specs=pl.BlockSpec((1, 8, g), lambda i: (i, 0, 0)),
        out_shape=jax.ShapeDtypeStruct((nsteps, 8, g), jnp.float32),
        interpret=interpret,
    )(targets, p)
    return jnp.sum(out, axis=(0, 2))


def _fcos_loss_pallas(p3, p4, p5, p6, p7, targets, interpret=False):
    tg = jnp.asarray(targets, jnp.float32)
    acc = None
    for li, p in enumerate((p3, p4, p5, p6, p7)):
        part = _run_level(p, tg, li, interpret=interpret)
        acc = part if acc is None else acc + part
    n = acc[3]
    lbox = acc[0] / n
    lcnt = acc[1] / n
    lcls = acc[2] / n
    loss = lbox + lcnt + lcls
    return (loss, lbox, lcnt, lcls)


def kernel(p3, p4, p5, p6, p7, targets, image_size):
    return _fcos_loss_pallas(p3, p4, p5, p6, p7, targets)
